# baseline (speedup 1.0000x reference)
import functools

import jax
import jax.numpy as jnp
from jax import lax
from jax.experimental import pallas as pl
from jax.experimental.pallas import tpu as pltpu
from jax.experimental.pallas import tpu_sc as plsc

_NUM_CORES = 2
_NUM_SUBCORES = 16
_NUM_WORKERS = _NUM_CORES * _NUM_SUBCORES

_TILESPMEM_BYTES = 131071 * 4


def _pick_chunks(rows_per_worker, row_bytes):
    max_rows = max(8, (_TILESPMEM_BYTES // 2) // row_bytes // 8 * 8)
    big = min(rows_per_worker, max_rows)
    chunks = [big] * (rows_per_worker // big)
    if rows_per_worker % big:
        chunks.append(rows_per_worker % big)
    return chunks


@functools.cache
def _make_scs_broadcast(batch, seq, d_model, dtype):
    rows_per_core = seq // _NUM_CORES
    row_bytes = d_model * dtype.itemsize
    spmem_budget = 8188 * 1024
    max_rows = (spmem_budget // 2) // row_bytes // 8 * 8
    big = min(rows_per_core, max_rows)
    chunks = [big] * (rows_per_core // big)
    if rows_per_core % big:
        chunks.append(rows_per_core % big)
    starts = [sum(chunks[:i]) for i in range(len(chunks))]
    nchunks = len(chunks)
    buf_rows = max(chunks)

    mesh = plsc.ScalarSubcoreMesh(axis_name="c", num_cores=_NUM_CORES)

    @functools.partial(
        pl.kernel,
        out_type=jax.ShapeDtypeStruct((batch * seq, d_model), dtype),
        mesh=mesh,
        scratch_types=[
            pltpu.VMEM_SHARED((buf_rows, d_model), dtype),
            pltpu.VMEM_SHARED((buf_rows, d_model), dtype),
            pltpu.SemaphoreType.DMA,
            pltpu.SemaphoreType.DMA,
            pltpu.SemaphoreType.DMA,
            pltpu.SemaphoreType.DMA,
        ],
    )
    def pe_broadcast(table_hbm, out_hbm, buf0, buf1, isem0, isem1, osem0, osem1):
        cid = lax.axis_index("c")
        base = cid * rows_per_core
        bufs = (buf0, buf1)
        isems = (isem0, isem1)
        osems = (osem0, osem1)

        in_copies = [None] * nchunks
        out_copies = [[] for _ in range(nchunks)]

        def start_in(i):
            in_copies[i] = pltpu.async_copy(
                table_hbm.at[pl.ds(base + starts[i], chunks[i])],
                bufs[i % 2].at[pl.ds(0, chunks[i])],
                isems[i % 2],
            )

        start_in(0)
        for i in range(nchunks):
            in_copies[i].wait()
            for b in range(batch):
                out_copies[i].append(
                    pltpu.async_copy(
                        bufs[i % 2].at[pl.ds(0, chunks[i])],
                        out_hbm.at[pl.ds(b * seq + base + starts[i], chunks[i])],
                        osems[i % 2],
                    )
                )
            if i + 1 < nchunks:
                if i >= 1:
                    for c in out_copies[i - 1]:
                        c.wait()
                start_in(i + 1)
        if nchunks >= 2:
            for c in out_copies[nchunks - 2]:
                c.wait()
        for c in out_copies[nchunks - 1]:
            c.wait()

    return pe_broadcast


@functools.cache
def _make_sc_broadcast(batch, seq, d_model, dtype):
    rows_per_worker = seq // _NUM_WORKERS
    chunks = _pick_chunks(rows_per_worker, d_model * dtype.itemsize)
    starts = [sum(chunks[:i]) for i in range(len(chunks))]
    nchunks = len(chunks)
    buf_rows = max(chunks)

    mesh = plsc.VectorSubcoreMesh(
        core_axis_name="c",
        subcore_axis_name="s",
        num_cores=_NUM_CORES,
        num_subcores=_NUM_SUBCORES,
    )

    @functools.partial(
        pl.kernel,
        out_type=jax.ShapeDtypeStruct((batch * seq, d_model), dtype),
        mesh=mesh,
        scratch_types=[
            pltpu.VMEM((buf_rows, d_model), dtype),
            pltpu.VMEM((buf_rows, d_model), dtype),
            pltpu.SemaphoreType.DMA,
            pltpu.SemaphoreType.DMA,
            pltpu.SemaphoreType.DMA,
            pltpu.SemaphoreType.DMA,
        ],
    )
    def pe_broadcast(table_hbm, out_hbm, buf0, buf1, isem0, isem1, osem0, osem1):
        wid = lax.axis_index("s") * _NUM_CORES + lax.axis_index("c")
        base = wid * rows_per_worker
        bufs = (buf0, buf1)
        isems = (isem0, isem1)
        osems = (osem0, osem1)

        in_copies = [None] * nchunks
        out_copies = [[] for _ in range(nchunks)]

        def start_in(i):
            in_copies[i] = pltpu.async_copy(
                table_hbm.at[pl.ds(base + starts[i], chunks[i])],
                bufs[i % 2].at[pl.ds(0, chunks[i])],
                isems[i % 2],
            )

        start_in(0)
        for i in range(nchunks):
            in_copies[i].wait()
            for b in range(batch):
                out_copies[i].append(
                    pltpu.async_copy(
                        bufs[i % 2].at[pl.ds(0, chunks[i])],
                        out_hbm.at[pl.ds(b * seq + base + starts[i], chunks[i])],
                        osems[i % 2],
                    )
                )
            if i + 1 < nchunks:
                if i >= 1:
                    for c in out_copies[i - 1]:
                        c.wait()
                start_in(i + 1)
        if nchunks >= 2:
            for c in out_copies[nchunks - 2]:
                c.wait()
        for c in out_copies[nchunks - 1]:
            c.wait()

    return pe_broadcast


def kernel(x, pos_embed_weight):
    batch, seq, d_model = x.shape
    assert seq % _NUM_WORKERS == 0, seq
    table = pos_embed_weight[:seq] if pos_embed_weight.shape[0] != seq else pos_embed_weight
    fn = _make_scs_broadcast(batch, seq, d_model, jnp.dtype(table.dtype))
    out_flat = fn(table)
    return out_flat.reshape(batch, seq, d_model)


# speedup vs baseline: 1.2113x; 1.2113x over previous
import functools

import jax
import jax.numpy as jnp
from jax import lax
from jax.experimental import pallas as pl
from jax.experimental.pallas import tpu as pltpu
from jax.experimental.pallas import tpu_sc as plsc

_NUM_CORES = 2
_NUM_SUBCORES = 16
_NUM_WORKERS = _NUM_CORES * _NUM_SUBCORES

_TILESPMEM_BYTES = 131071 * 4


def _pick_chunks(rows_per_worker, row_bytes):
    max_rows = max(8, (_TILESPMEM_BYTES // 2) // row_bytes // 8 * 8)
    big = min(rows_per_worker, max_rows)
    chunks = []
    if rows_per_worker > 2 * 8:
        chunks.append(8)
    rem = rows_per_worker - sum(chunks)
    chunks.extend([big] * (rem // big))
    if rem % big:
        chunks.append(rem % big)
    return chunks


@functools.cache
def _make_sc_broadcast(batch, seq, d_model, dtype):
    rows_per_worker = seq // _NUM_WORKERS
    chunks = _pick_chunks(rows_per_worker, d_model * dtype.itemsize)
    starts = [sum(chunks[:i]) for i in range(len(chunks))]
    nchunks = len(chunks)
    buf_rows = max(chunks)

    mesh = plsc.VectorSubcoreMesh(
        core_axis_name="c",
        subcore_axis_name="s",
        num_cores=_NUM_CORES,
        num_subcores=_NUM_SUBCORES,
    )

    @functools.partial(
        pl.kernel,
        out_type=jax.ShapeDtypeStruct((batch * seq, d_model), dtype),
        mesh=mesh,
        scratch_types=[
            pltpu.VMEM((buf_rows, d_model), dtype),
            pltpu.VMEM((buf_rows, d_model), dtype),
            pltpu.SemaphoreType.DMA,
            pltpu.SemaphoreType.DMA,
            pltpu.SemaphoreType.DMA,
            pltpu.SemaphoreType.DMA,
        ],
    )
    def pe_broadcast(table_hbm, out_hbm, buf0, buf1, isem0, isem1, osem0, osem1):
        wid = lax.axis_index("s") * _NUM_CORES + lax.axis_index("c")
        base = wid * rows_per_worker
        bufs = (buf0, buf1)
        isems = (isem0, isem1)
        osems = (osem0, osem1)

        in_copies = [None] * nchunks
        out_copies = [[] for _ in range(nchunks)]

        def start_in(i):
            in_copies[i] = pltpu.async_copy(
                table_hbm.at[pl.ds(base + starts[i], chunks[i])],
                bufs[i % 2].at[pl.ds(0, chunks[i])],
                isems[i % 2],
            )

        start_in(0)
        for i in range(nchunks):
            in_copies[i].wait()
            for b in range(batch):
                out_copies[i].append(
                    pltpu.async_copy(
                        bufs[i % 2].at[pl.ds(0, chunks[i])],
                        out_hbm.at[pl.ds(b * seq + base + starts[i], chunks[i])],
                        osems[i % 2],
                    )
                )
            if i + 1 < nchunks:
                if i >= 1:
                    for c in out_copies[i - 1]:
                        c.wait()
                start_in(i + 1)
        if nchunks >= 2:
            for c in out_copies[nchunks - 2]:
                c.wait()
        for c in out_copies[nchunks - 1]:
            c.wait()

    return pe_broadcast


def kernel(x, pos_embed_weight):
    batch, seq, d_model = x.shape
    assert seq % _NUM_WORKERS == 0, seq
    table = pos_embed_weight[:seq] if pos_embed_weight.shape[0] != seq else pos_embed_weight
    fn = _make_sc_broadcast(batch, seq, d_model, jnp.dtype(table.dtype))
    out_flat = fn(table)
    return out_flat.reshape(batch, seq, d_model)


# speedup vs baseline: 1.2186x; 1.0060x over previous
import functools

import jax
import jax.numpy as jnp
from jax import lax
from jax.experimental import pallas as pl
from jax.experimental.pallas import tpu as pltpu
from jax.experimental.pallas import tpu_sc as plsc

_NUM_CORES = 2
_NUM_SUBCORES = 16
_NUM_WORKERS = _NUM_CORES * _NUM_SUBCORES

_TILESPMEM_BYTES = 131071 * 4


def _pick_chunks(rows_per_worker, row_bytes):
    max_rows = max(8, (_TILESPMEM_BYTES // 2) // row_bytes // 8 * 8)
    big = min(rows_per_worker, max_rows)
    chunks = [big] * (rows_per_worker // big)
    if rows_per_worker % big:
        chunks.append(rows_per_worker % big)
    return chunks


@functools.cache
def _make_sc_broadcast(batch, seq, d_model, dtype):
    rows_per_worker = seq // _NUM_WORKERS
    chunks = _pick_chunks(rows_per_worker, d_model * dtype.itemsize)
    starts = [sum(chunks[:i]) for i in range(len(chunks))]
    nchunks = len(chunks)
    buf_rows = max(chunks)

    mesh = plsc.VectorSubcoreMesh(
        core_axis_name="c",
        subcore_axis_name="s",
        num_cores=_NUM_CORES,
        num_subcores=_NUM_SUBCORES,
    )

    @functools.partial(
        pl.kernel,
        out_type=jax.ShapeDtypeStruct((batch * seq, d_model), dtype),
        mesh=mesh,
        scratch_types=[
            pltpu.VMEM((buf_rows, d_model), dtype),
            pltpu.VMEM((buf_rows, d_model), dtype),
            pltpu.SemaphoreType.DMA,
            pltpu.SemaphoreType.DMA,
            pltpu.SemaphoreType.DMA,
            pltpu.SemaphoreType.DMA,
        ],
    )
    def pe_broadcast(table_hbm, out_hbm, buf0, buf1, isem0, isem1, osem0, osem1):
        wid = lax.axis_index("s") * _NUM_CORES + lax.axis_index("c")
        base = wid * rows_per_worker
        bufs = (buf0, buf1)
        isems = (isem0, isem1)
        osems = (osem0, osem1)

        in_copies = [None] * nchunks
        out_copies = [[] for _ in range(nchunks)]

        def start_in(i):
            in_copies[i] = pltpu.async_copy(
                table_hbm.at[pl.ds(base + starts[i], chunks[i])],
                bufs[i % 2].at[pl.ds(0, chunks[i])],
                isems[i % 2],
            )

        start_in(0)
        for i in range(nchunks):
            in_copies[i].wait()
            for b in range(batch):
                out_copies[i].append(
                    pltpu.async_copy(
                        bufs[i % 2].at[pl.ds(0, chunks[i])],
                        out_hbm.at[pl.ds(b * seq + base + starts[i], chunks[i])],
                        osems[i % 2],
                    )
                )
            if i + 1 < nchunks:
                if i >= 1:
                    for c in out_copies[i - 1]:
                        c.wait()
                start_in(i + 1)
        if nchunks >= 2:
            for c in out_copies[nchunks - 2]:
                c.wait()
        for c in out_copies[nchunks - 1]:
            c.wait()

    return pe_broadcast


def kernel(x, pos_embed_weight):
    batch, seq, d_model = x.shape
    assert seq % _NUM_WORKERS == 0, seq
    table = pos_embed_weight[:seq] if pos_embed_weight.shape[0] != seq else pos_embed_weight
    fn = _make_sc_broadcast(batch, seq, d_model, jnp.dtype(table.dtype))
    out_flat = fn(table)
    return out_flat.reshape(batch, seq, d_model)
